# Initial kernel scaffold; baseline (speedup 1.0000x reference)
import jax
import jax.numpy as jnp
from jax.experimental import pallas as pl


def kernel(x):
    raise NotImplementedError("write your pallas kernel here")


# speedup vs baseline: 6.8991x; 6.8991x over previous
import functools

import jax
import jax.numpy as jnp
from jax.experimental import pallas as pl
from jax.experimental.pallas import tpu as pltpu


def _unfold(x):
    B, C, H, W = x.shape
    xp = jnp.pad(x, ((0, 0), (0, 0), (1, 1), (1, 1)))
    patches = [xp[:, :, ki:ki + H, kj:kj + W] for ki in range(3) for kj in range(3)]
    p = jnp.stack(patches, axis=2)
    return p.reshape(B, C * 9, H * W)


def _body(xbT_ref, xbcol_ref, q_ref, out_ref, *, TILE, HW, C):
    NK = 9

    xbT = xbT_ref[0]
    xbcol = xbcol_ref[0]
    Rt = jax.lax.dot_general(xbT, xbcol, (((1,), (0,)), ((), ())),
                             preferred_element_type=jnp.float32)

    coln = jax.lax.broadcasted_iota(jnp.int32, (HW, TILE), 0)
    rowi = jax.lax.broadcasted_iota(jnp.int32, (TILE, HW), 1)
    q = q_ref[0]
    scale = 1.0 / (C ** 0.5)

    gn_list = []
    sc_list = []
    v = Rt
    for j in range(3):
        m = jnp.max(v, axis=0, keepdims=True)
        idx = jnp.min(jnp.where(v >= m, coln, HW), axis=0, keepdims=True)
        oh = coln == idx
        v = jnp.where(oh, -jnp.inf, v)
        ohT = (rowi == idx.T).astype(jnp.bfloat16)
        for a in range(NK):
            tab = xbT_ref[0, :, pl.ds(a * C, C)]
            g = jax.lax.dot_general(ohT, tab, (((1,), (0,)), ((), ())),
                                    preferred_element_type=jnp.float32)
            gn_list.append(g)
            sc_list.append(jnp.sum(g * q, axis=1, keepdims=True) * scale)

    S = jnp.concatenate(sc_list, axis=1)
    mS = jnp.max(S, axis=1, keepdims=True)
    E = jnp.exp(S - mS)
    W = E / jnp.sum(E, axis=1, keepdims=True)

    acc = jnp.zeros((TILE, C), jnp.float32)
    for k in range(27):
        acc = acc + W[:, k:k + 1] * gn_list[k]
    out_ref[0] = acc


@jax.jit
def kernel(x):
    B, C, H, W = x.shape
    HW = H * W
    xu = _unfold(x)
    norm = jnp.sqrt(jnp.sum(xu * xu, axis=2, keepdims=True))
    xun = xu / jnp.maximum(norm, 1e-12)
    xb = xun.astype(jnp.bfloat16)
    xbT = xb.transpose(0, 2, 1)
    q = x.reshape(B, C, HW).transpose(0, 2, 1)

    TILE = 128 if HW % 128 == 0 else HW
    grid = (B, HW // TILE)

    out = pl.pallas_call(
        functools.partial(_body, TILE=TILE, HW=HW, C=C),
        grid=grid,
        in_specs=[
            pl.BlockSpec((1, HW, 9 * C), lambda b, t: (b, 0, 0)),
            pl.BlockSpec((1, 9 * C, TILE), lambda b, t: (b, 0, t)),
            pl.BlockSpec((1, TILE, C), lambda b, t: (b, t, 0)),
        ],
        out_specs=pl.BlockSpec((1, TILE, C), lambda b, t: (b, t, 0)),
        out_shape=jax.ShapeDtypeStruct((B, HW, C), jnp.float32),
        compiler_params=pltpu.CompilerParams(
            dimension_semantics=("parallel", "arbitrary"),
        ),
    )(xbT, xb, q)
    return out


# speedup vs baseline: 6.9065x; 1.0011x over previous
import functools

import jax
import jax.numpy as jnp
from jax.experimental import pallas as pl
from jax.experimental.pallas import tpu as pltpu


def _unfold(x):
    B, C, H, W = x.shape
    xp = jnp.pad(x, ((0, 0), (0, 0), (1, 1), (1, 1)))
    patches = [xp[:, :, ki:ki + H, kj:kj + W] for ki in range(3) for kj in range(3)]
    p = jnp.stack(patches, axis=2)
    return p.reshape(B, C * 9, H * W)


def _body(xbT_ref, q_ref, out_ref, *, TILE, HW, C):
    NK = 9
    t = pl.program_id(1)

    xbT = xbT_ref[0]
    rows = xbT_ref[0, pl.ds(t * TILE, TILE), :]
    xbcol = rows.T
    Rt = jax.lax.dot_general(xbT, xbcol, (((1,), (0,)), ((), ())),
                             preferred_element_type=jnp.float32)

    coln = jax.lax.broadcasted_iota(jnp.int32, (HW, TILE), 0)
    rowi = jax.lax.broadcasted_iota(jnp.int32, (TILE, HW), 1)
    q = q_ref[0]
    scale = 1.0 / (C ** 0.5)

    gn_list = []
    sc_list = []
    v = Rt
    for j in range(3):
        m = jnp.max(v, axis=0, keepdims=True)
        idx = jnp.min(jnp.where(v >= m, coln, HW), axis=0, keepdims=True)
        oh = coln == idx
        v = jnp.where(oh, -jnp.inf, v)
        ohT = (rowi == idx.T).astype(jnp.bfloat16)
        for a in range(NK):
            tab = xbT_ref[0, :, pl.ds(a * C, C)]
            g = jax.lax.dot_general(ohT, tab, (((1,), (0,)), ((), ())),
                                    preferred_element_type=jnp.float32)
            gn_list.append(g)
            sc_list.append(jnp.sum(g * q, axis=1, keepdims=True) * scale)

    S = jnp.concatenate(sc_list, axis=1)
    mS = jnp.max(S, axis=1, keepdims=True)
    E = jnp.exp(S - mS)
    W = E / jnp.sum(E, axis=1, keepdims=True)

    acc = jnp.zeros((TILE, C), jnp.float32)
    for k in range(27):
        acc = acc + W[:, k:k + 1] * gn_list[k]
    out_ref[0] = acc


@jax.jit
def kernel(x):
    B, C, H, W = x.shape
    HW = H * W
    xu = _unfold(x)
    norm = jnp.sqrt(jnp.sum(xu * xu, axis=2, keepdims=True))
    xun = xu / jnp.maximum(norm, 1e-12)
    xb = xun.astype(jnp.bfloat16)
    xbT = xb.transpose(0, 2, 1)
    q = x.reshape(B, C, HW).transpose(0, 2, 1)

    TILE = 128 if HW % 128 == 0 else HW
    grid = (B, HW // TILE)

    out = pl.pallas_call(
        functools.partial(_body, TILE=TILE, HW=HW, C=C),
        grid=grid,
        in_specs=[
            pl.BlockSpec((1, HW, 9 * C), lambda b, t: (b, 0, 0)),
            pl.BlockSpec((1, TILE, C), lambda b, t: (b, t, 0)),
        ],
        out_specs=pl.BlockSpec((1, TILE, C), lambda b, t: (b, t, 0)),
        out_shape=jax.ShapeDtypeStruct((B, HW, C), jnp.float32),
        compiler_params=pltpu.CompilerParams(
            dimension_semantics=("parallel", "arbitrary"),
        ),
    )(xbT, q)
    return out


# speedup vs baseline: 8.6074x; 1.2463x over previous
import functools

import jax
import jax.numpy as jnp
from jax.experimental import pallas as pl
from jax.experimental.pallas import tpu as pltpu


def _unfold(x):
    B, C, H, W = x.shape
    xp = jnp.pad(x, ((0, 0), (0, 0), (1, 1), (1, 1)))
    patches = [xp[:, :, ki:ki + H, kj:kj + W] for ki in range(3) for kj in range(3)]
    p = jnp.stack(patches, axis=2)
    return p.reshape(B, C * 9, H * W)


def _body(xbT_ref, q_ref, out_ref, *, TILE, HW, C):
    NK = 9
    t = pl.program_id(1)

    xbT = xbT_ref[0]
    rows = xbT_ref[0, pl.ds(t * TILE, TILE), :]
    xbcol = rows.T
    Rt = jax.lax.dot_general(xbT, xbcol, (((1,), (0,)), ((), ())),
                             preferred_element_type=jnp.float32)

    coln = jax.lax.broadcasted_iota(jnp.int32, (HW, TILE), 0)
    rowi = jax.lax.broadcasted_iota(jnp.int32, (TILE, HW), 1)
    q = q_ref[0]
    scale = 1.0 / (C ** 0.5)

    gn_list = []
    sc_list = []
    v = Rt
    for j in range(3):
        m = jnp.max(v, axis=0, keepdims=True)
        idx = jnp.min(jnp.where(v >= m, coln, HW), axis=0, keepdims=True)
        oh = coln == idx
        v = jnp.where(oh, -jnp.inf, v)
        ohT = (rowi == idx.T).astype(jnp.bfloat16)
        for a in range(NK):
            tab = xbT_ref[0, :, pl.ds(a * C, C)]
            g = jax.lax.dot_general(ohT, tab, (((1,), (0,)), ((), ())),
                                    preferred_element_type=jnp.float32)
            gn_list.append(g)
            sc_list.append(jnp.sum(g * q, axis=1, keepdims=True) * scale)

    S = jnp.concatenate(sc_list, axis=1)
    mS = jnp.max(S, axis=1, keepdims=True)
    E = jnp.exp(S - mS)
    W = E / jnp.sum(E, axis=1, keepdims=True)

    acc = jnp.zeros((TILE, C), jnp.float32)
    for k in range(27):
        acc = acc + W[:, k:k + 1] * gn_list[k]
    out_ref[0] = acc


@jax.jit
def kernel(x):
    B, C, H, W = x.shape
    HW = H * W
    xu = _unfold(x)
    norm = jnp.sqrt(jnp.sum(xu * xu, axis=2, keepdims=True))
    xun = xu / jnp.maximum(norm, 1e-12)
    xb = xun.astype(jnp.bfloat16)
    xbT = xb.transpose(0, 2, 1)
    q = x.reshape(B, C, HW).transpose(0, 2, 1)

    TILE = 256 if HW % 256 == 0 else HW
    grid = (B, HW // TILE)

    out = pl.pallas_call(
        functools.partial(_body, TILE=TILE, HW=HW, C=C),
        grid=grid,
        in_specs=[
            pl.BlockSpec((1, HW, 9 * C), lambda b, t: (b, 0, 0)),
            pl.BlockSpec((1, TILE, C), lambda b, t: (b, t, 0)),
        ],
        out_specs=pl.BlockSpec((1, TILE, C), lambda b, t: (b, t, 0)),
        out_shape=jax.ShapeDtypeStruct((B, HW, C), jnp.float32),
        compiler_params=pltpu.CompilerParams(
            dimension_semantics=("parallel", "arbitrary"),
        ),
    )(xbT, q)
    return out
